# baseline (speedup 1.0000x reference)
import jax
import jax.numpy as jnp
from jax.experimental import pallas as pl
from jax.experimental.pallas import tpu as pltpu

_THRESH = 0.5
_RATIO = 3.0
_V0 = 0.1
_V1 = 0.2
_BISECT_ITERS = 24


def _make_batch_kernel(B, P, C, O):
    def _batch_kernel(loc_ref, conf_ref, pri_ref, tb_ref, tl_ref,
                      part_ref, v_ref):
        pri = pri_ref[...]
        pcx, pcy, pw, ph = pri[0], pri[1], pri[2], pri[3]
        px1 = pcx - pw / 2.0
        py1 = pcy - ph / 2.0
        px2 = pcx + pw / 2.0
        py2 = pcy + ph / 2.0
        tbc = tb_ref[0]
        tx1, ty1 = tbc[:, 0:1], tbc[:, 1:2]
        tx2, ty2 = tbc[:, 2:3], tbc[:, 3:4]
        lblc = tl_ref[0].astype(jnp.float32)

        iw = jnp.maximum(
            jnp.minimum(tx2, px2[None, :]) - jnp.maximum(tx1, px1[None, :]),
            0.0)
        ih = jnp.maximum(
            jnp.minimum(ty2, py2[None, :]) - jnp.maximum(ty1, py1[None, :]),
            0.0)
        inter = iw * ih
        area_t = (tx2 - tx1) * (ty2 - ty1)
        area_p = ((px2 - px1) * (py2 - py1))[None, :]
        ov = inter / (area_t + area_p - inter)

        jj = jax.lax.broadcasted_iota(jnp.int32, (O, P), 0)
        pp = jax.lax.broadcasted_iota(jnp.int32, (O, P), 1)

        bt_ov = jnp.max(ov, axis=0)
        bt_idx = jnp.min(jnp.where(ov == bt_ov[None, :], jj, O), axis=0)
        mx = jnp.max(ov, axis=1, keepdims=True)
        bpi = jnp.min(jnp.where(ov == mx, pp, P), axis=1, keepdims=True)

        eq = bpi == pp
        fj = jnp.max(jnp.where(eq, jj, -1), axis=0)
        fidx = jnp.where(fj >= 0, fj, bt_idx)
        fov = jnp.where(fj >= 0, 2.0, bt_ov)

        oh = (fidx[None, :] == jj).astype(jnp.float32)
        coef = jnp.concatenate(
            [tbc, lblc, jnp.zeros((O, 3), jnp.float32)], axis=1)
        gath = jax.lax.dot_general(
            coef, oh, (((0,), (0,)), ((), ())),
            preferred_element_type=jnp.float32)

        lblg = gath[4]
        confl = jnp.where(fov < _THRESH, 0.0, lblg + 1.0)
        posf = (confl > 0.0).astype(jnp.float32)

        M1 = gath[0:2]
        M2 = gath[2:4]
        PC = pri[0:2]
        PW = pri[2:4]
        Gc = ((M1 + M2) / 2.0 - PC) / (_V0 * PW)
        Gw = jnp.log((M2 - M1) / PW) / _V1
        G = jnp.concatenate([Gc, Gw], axis=0)
        d = loc_ref[0] - G
        ad = jnp.abs(d)
        sl1 = jnp.where(ad < 1.0, 0.5 * d * d, ad - 0.5)
        ll = jnp.sum(jnp.sum(sl1, axis=0) * posf)

        cf = conf_ref[0].T
        m = jnp.max(cf, axis=0)
        e = jnp.exp(cf - m[None, :])
        lbl = confl.astype(jnp.int32)
        ccs = jax.lax.broadcasted_iota(jnp.int32, (C, P), 0)
        sel = jnp.where(ccs == lbl[None, :], cf, 0.0)
        ri = jax.lax.broadcasted_iota(jnp.int32, (8, C), 0)
        onesC = jnp.where(ri == 0, 1.0, 0.0).astype(jnp.float32)
        s = jax.lax.dot_general(
            onesC, e, (((1,), (0,)), ((), ())),
            preferred_element_type=jnp.float32)[0]
        xl = jax.lax.dot_general(
            onesC, sel, (((1,), (0,)), ((), ())),
            preferred_element_type=jnp.float32)[0]
        lse = jnp.log(s) + m
        loss_c = lse - xl

        v_ref[0, 0, :] = loss_c * (1.0 - posf)

        lcp = jnp.sum(loss_c * posf)
        npos = jnp.sum(posf)
        pr = jax.lax.broadcasted_iota(jnp.int32, (8, 128), 0)
        out8 = jnp.where(
            pr == 0, ll, jnp.where(pr == 1, lcp,
                                   jnp.where(pr == 2, npos, 0.0)))
        part_ref[0] = out8.astype(jnp.float32)

    return _batch_kernel


def _make_neg_kernel(B, P):
    def _neg_kernel(part_ref, v_ref, out_l, out_c):
        V = v_ref[...]
        pt = part_ref[...]
        ll_b = pt[:, 0, 0:1]
        lcp_b = pt[:, 1, 0:1]
        npos_b = pt[:, 2, 0:1]
        k = jnp.minimum(_RATIO * npos_b, jnp.float32(P - 1))
        mv = jnp.max(V, axis=1, keepdims=True)

        def body(_, lh):
            lo, hi = lh
            mid = 0.5 * (lo + hi)
            cnt = jnp.sum((V > mid).astype(jnp.float32), axis=1,
                          keepdims=True)
            big = cnt > k
            return (jnp.where(big, mid, lo), jnp.where(big, hi, mid))

        _, hi = jax.lax.fori_loop(
            0, _BISECT_ITERS, body,
            (jnp.full((B, 1), -1.0, jnp.float32), mv))
        mask = (V > hi).astype(jnp.float32)
        cnt_hi = jnp.sum(mask, axis=1, keepdims=True)
        sneg = jnp.sum(V * mask, axis=1, keepdims=True) + (k - cnt_hi) * hi

        N = jnp.sum(npos_b)
        out_l[...] = jnp.sum(ll_b).reshape(1, 1) / N
        out_c[...] = (jnp.sum(lcp_b) + jnp.sum(sneg)).reshape(1, 1) / N

    return _neg_kernel


def kernel(pred_loc, pred_conf, priors, target_boxes, target_labels):
    B, P, _ = pred_loc.shape
    C = pred_conf.shape[-1]
    O = target_boxes.shape[1]

    loc_t = jnp.transpose(pred_loc, (0, 2, 1))
    tl3 = target_labels.reshape(B, O, 1).astype(jnp.int32)
    pri_t = priors.T

    part, vmat = pl.pallas_call(
        _make_batch_kernel(B, P, C, O),
        grid=(B,),
        in_specs=[
            pl.BlockSpec((1, 4, P), lambda b: (b, 0, 0)),
            pl.BlockSpec((1, P, C), lambda b: (b, 0, 0)),
            pl.BlockSpec((4, P), lambda b: (0, 0)),
            pl.BlockSpec((1, O, 4), lambda b: (b, 0, 0)),
            pl.BlockSpec((1, O, 1), lambda b: (b, 0, 0)),
        ],
        out_specs=[
            pl.BlockSpec((1, 8, 128), lambda b: (b, 0, 0)),
            pl.BlockSpec((1, 1, P), lambda b: (b, 0, 0)),
        ],
        out_shape=[
            jax.ShapeDtypeStruct((B, 8, 128), jnp.float32),
            jax.ShapeDtypeStruct((B, 1, P), jnp.float32),
        ],
        compiler_params=pltpu.CompilerParams(
            dimension_semantics=("parallel",)),
    )(loc_t, pred_conf, pri_t, target_boxes, tl3)

    out_l, out_c = pl.pallas_call(
        _make_neg_kernel(B, P),
        out_shape=[jax.ShapeDtypeStruct((1, 1), jnp.float32)] * 2,
    )(part, vmat.reshape(B, P))

    return (out_l[0, 0], out_c[0, 0])


# speedup vs baseline: 1.7186x; 1.7186x over previous
import jax
import jax.numpy as jnp
from jax.experimental import pallas as pl
from jax.experimental.pallas import tpu as pltpu

_THRESH = 0.5
_RATIO = 3.0
_V0 = 0.1
_V1 = 0.2
_BISECT_ITERS = 24


def _make_batch_kernel(B, P, C, O):
    def _batch_kernel(loc_ref, conf_ref, pri_ref, tb_ref, tl_ref,
                      part_ref, v_ref):
        pri = pri_ref[...]
        pcx, pcy, pw, ph = pri[0], pri[1], pri[2], pri[3]
        px1 = pcx - pw / 2.0
        py1 = pcy - ph / 2.0
        px2 = pcx + pw / 2.0
        py2 = pcy + ph / 2.0
        tbc = tb_ref[0]
        tx1, ty1 = tbc[:, 0:1], tbc[:, 1:2]
        tx2, ty2 = tbc[:, 2:3], tbc[:, 3:4]
        lblc = tl_ref[0].astype(jnp.float32)

        iw = jnp.maximum(
            jnp.minimum(tx2, px2[None, :]) - jnp.maximum(tx1, px1[None, :]),
            0.0)
        ih = jnp.maximum(
            jnp.minimum(ty2, py2[None, :]) - jnp.maximum(ty1, py1[None, :]),
            0.0)
        inter = iw * ih
        area_t = (tx2 - tx1) * (ty2 - ty1)
        area_p = ((px2 - px1) * (py2 - py1))[None, :]
        ov = inter / (area_t + area_p - inter)

        jj = jax.lax.broadcasted_iota(jnp.int32, (O, P), 0)
        pp = jax.lax.broadcasted_iota(jnp.int32, (O, P), 1)

        bt_ov = jnp.max(ov, axis=0)
        bt_idx = jnp.min(jnp.where(ov == bt_ov[None, :], jj, O), axis=0)
        mx = jnp.max(ov, axis=1, keepdims=True)
        bpi = jnp.min(jnp.where(ov == mx, pp, P), axis=1, keepdims=True)

        eq = bpi == pp
        fj = jnp.max(jnp.where(eq, jj, -1), axis=0)
        fidx = jnp.where(fj >= 0, fj, bt_idx)
        fov = jnp.where(fj >= 0, 2.0, bt_ov)

        oh = (fidx[None, :] == jj).astype(jnp.float32)
        coef = jnp.concatenate(
            [tbc, lblc, jnp.zeros((O, 3), jnp.float32)], axis=1)
        gath = jax.lax.dot_general(
            coef, oh, (((0,), (0,)), ((), ())),
            preferred_element_type=jnp.float32)

        lblg = gath[4]
        confl = jnp.where(fov < _THRESH, 0.0, lblg + 1.0)
        posf = (confl > 0.0).astype(jnp.float32)

        M1 = gath[0:2]
        M2 = gath[2:4]
        PC = pri[0:2]
        PW = pri[2:4]
        Gc = ((M1 + M2) / 2.0 - PC) / (_V0 * PW)
        Gw = jnp.log((M2 - M1) / PW) / _V1
        G = jnp.concatenate([Gc, Gw], axis=0)
        d = loc_ref[0] - G
        ad = jnp.abs(d)
        sl1 = jnp.where(ad < 1.0, 0.5 * d * d, ad - 0.5)
        ll = jnp.sum(jnp.sum(sl1, axis=0) * posf)

        cf = conf_ref[0]
        m = jnp.max(cf, axis=0)
        e = jnp.exp(cf - m[None, :])
        lbl = confl.astype(jnp.int32)
        ccs = jax.lax.broadcasted_iota(jnp.int32, (C, P), 0)
        sel = jnp.where(ccs == lbl[None, :], cf, 0.0)
        ri = jax.lax.broadcasted_iota(jnp.int32, (8, C), 0)
        onesC = jnp.where(ri == 0, 1.0, 0.0).astype(jnp.float32)
        s = jax.lax.dot_general(
            onesC, e, (((1,), (0,)), ((), ())),
            preferred_element_type=jnp.float32)[0]
        xl = jax.lax.dot_general(
            onesC, sel, (((1,), (0,)), ((), ())),
            preferred_element_type=jnp.float32)[0]
        lse = jnp.log(s) + m
        loss_c = lse - xl

        v_ref[0, 0, :] = loss_c * (1.0 - posf)

        lcp = jnp.sum(loss_c * posf)
        npos = jnp.sum(posf)
        pr = jax.lax.broadcasted_iota(jnp.int32, (8, 128), 0)
        out8 = jnp.where(
            pr == 0, ll, jnp.where(pr == 1, lcp,
                                   jnp.where(pr == 2, npos, 0.0)))
        part_ref[0] = out8.astype(jnp.float32)

    return _batch_kernel


def _make_neg_kernel(B, P):
    def _neg_kernel(part_ref, v_ref, out_l, out_c):
        V = v_ref[...]
        pt = part_ref[...]
        ll_b = pt[:, 0, 0:1]
        lcp_b = pt[:, 1, 0:1]
        npos_b = pt[:, 2, 0:1]
        k = jnp.minimum(_RATIO * npos_b, jnp.float32(P - 1))
        mv = jnp.max(V, axis=1, keepdims=True)

        def body(_, lh):
            lo, hi = lh
            mid = 0.5 * (lo + hi)
            cnt = jnp.sum((V > mid).astype(jnp.float32), axis=1,
                          keepdims=True)
            big = cnt > k
            return (jnp.where(big, mid, lo), jnp.where(big, hi, mid))

        _, hi = jax.lax.fori_loop(
            0, _BISECT_ITERS, body,
            (jnp.full((B, 1), -1.0, jnp.float32), mv))
        mask = (V > hi).astype(jnp.float32)
        cnt_hi = jnp.sum(mask, axis=1, keepdims=True)
        sneg = jnp.sum(V * mask, axis=1, keepdims=True) + (k - cnt_hi) * hi

        N = jnp.sum(npos_b)
        out_l[...] = jnp.sum(ll_b).reshape(1, 1) / N
        out_c[...] = (jnp.sum(lcp_b) + jnp.sum(sneg)).reshape(1, 1) / N

    return _neg_kernel


def kernel(pred_loc, pred_conf, priors, target_boxes, target_labels):
    B, P, _ = pred_loc.shape
    C = pred_conf.shape[-1]
    O = target_boxes.shape[1]

    loc_t = jnp.transpose(pred_loc, (0, 2, 1))
    conf_t = jnp.transpose(pred_conf, (0, 2, 1))
    tl3 = target_labels.reshape(B, O, 1).astype(jnp.int32)
    pri_t = priors.T

    part, vmat = pl.pallas_call(
        _make_batch_kernel(B, P, C, O),
        grid=(B,),
        in_specs=[
            pl.BlockSpec((1, 4, P), lambda b: (b, 0, 0)),
            pl.BlockSpec((1, C, P), lambda b: (b, 0, 0)),
            pl.BlockSpec((4, P), lambda b: (0, 0)),
            pl.BlockSpec((1, O, 4), lambda b: (b, 0, 0)),
            pl.BlockSpec((1, O, 1), lambda b: (b, 0, 0)),
        ],
        out_specs=[
            pl.BlockSpec((1, 8, 128), lambda b: (b, 0, 0)),
            pl.BlockSpec((1, 1, P), lambda b: (b, 0, 0)),
        ],
        out_shape=[
            jax.ShapeDtypeStruct((B, 8, 128), jnp.float32),
            jax.ShapeDtypeStruct((B, 1, P), jnp.float32),
        ],
        compiler_params=pltpu.CompilerParams(
            dimension_semantics=("parallel",)),
    )(loc_t, conf_t, pri_t, target_boxes, tl3)

    out_l, out_c = pl.pallas_call(
        _make_neg_kernel(B, P),
        out_shape=[jax.ShapeDtypeStruct((1, 1), jnp.float32)] * 2,
    )(part, vmat.reshape(B, P))

    return (out_l[0, 0], out_c[0, 0])
